# Initial kernel scaffold; baseline (speedup 1.0000x reference)
import jax
import jax.numpy as jnp
from jax.experimental import pallas as pl


def kernel(x, edge_index, batch, W1, as1, ad1, b1, W2, as2, ad2, b2, W3, as3, ad3, b3, gn1w, gn1b, gn1m, gn2w, gn2b, gn2m, gn3w, gn3b, gn3m, gn4w, gn4b, gn4m, gn5w, gn5b, gn5m, A1Wi, A1Wr, A1b, A2Wi, A2Wr, A2b, Wl, bl):
    raise NotImplementedError("write your pallas kernel here")


# speedup vs baseline: 1.4615x; 1.4615x over previous
import functools

import jax
import jax.numpy as jnp
from jax.experimental import pallas as pl

N_PAD = 10240
NB = 8


def _pool_readout_kernel(h_ref, brow_ref, bcol_ref, wl_ref, bl_ref, out_ref):
    h = h_ref[...]
    brow = brow_ref[...]
    bcol = bcol_ref[...]
    gids = jax.lax.broadcasted_iota(jnp.int32, (NB, N_PAD), 0)
    m = brow == gids
    mf = m.astype(jnp.float32)
    cnt = jnp.maximum(jnp.sum(mf, axis=1, keepdims=True), 1.0)
    x_add = jnp.dot(mf, h, precision=jax.lax.Precision.HIGHEST,
                    preferred_element_type=jnp.float32)
    x_mean = x_add / cnt
    neg_inf = jnp.float32(-jnp.inf)
    maxes = []
    for g in range(NB):
        hm = jnp.where(bcol == g, h, neg_inf)
        maxes.append(jnp.max(hm, axis=0, keepdims=True))
    x_max = jnp.concatenate(maxes, axis=0)
    x_max = jnp.where(jnp.isfinite(x_max), x_max, 0.0)

    sel63 = (jax.lax.broadcasted_iota(jnp.int32, (1, 64), 1) == 63).astype(jnp.float32)
    key = jax.lax.dot_general(sel63, h, (((1,), (1,)), ((), ())),
                              precision=jax.lax.Precision.HIGHEST,
                              preferred_element_type=jnp.float32)
    masked = jnp.where(m, key, neg_inf)
    iota_n = jax.lax.broadcasted_iota(jnp.int32, (NB, N_PAD), 1)
    feats = []
    for _ in range(5):
        mx = jnp.max(masked, axis=1, keepdims=True)
        sel = (masked == mx) & jnp.isfinite(mx)
        idx = jnp.min(jnp.where(sel, iota_n, N_PAD), axis=1, keepdims=True)
        oh = (iota_n == idx).astype(jnp.float32)
        feats.append(jnp.dot(oh, h, precision=jax.lax.Precision.HIGHEST,
                             preferred_element_type=jnp.float32))
        masked = jnp.where(iota_n == idx, neg_inf, masked)
    x_aggr = jnp.concatenate(feats, axis=1)

    z = jnp.concatenate([x_max, x_mean, x_add, x_aggr], axis=1)
    zb = z.astype(jnp.bfloat16).astype(jnp.float32)
    wlb = wl_ref[...].astype(jnp.bfloat16).astype(jnp.float32)
    out_ref[...] = jnp.dot(zb, wlb, precision=jax.lax.Precision.HIGHEST,
                           preferred_element_type=jnp.float32) + bl_ref[...]


def _pool_readout(h, batch, Wl, bl):
    n = h.shape[0]
    h_pad = jnp.pad(h, ((0, N_PAD - n), (0, 0)))
    bpad = jnp.pad(batch, (0, N_PAD - n), constant_values=NB)
    return pl.pallas_call(
        _pool_readout_kernel,
        out_shape=jax.ShapeDtypeStruct((NB, 2), jnp.float32),
    )(h_pad, bpad[None, :], bpad[:, None], Wl, bl[None, :])


def _segsum(vals, seg, n):
    return jax.ops.segment_sum(vals, seg, n)


def _gat(xin, W, a_s, a_d, b, src, dst, n):
    h = jnp.dot(xin, W)
    s = (h * a_s).sum(-1)
    d = (h * a_d).sum(-1)
    e = jax.nn.leaky_relu(s[src] + d[dst], 0.2)
    ex = jnp.exp(e)
    den = _segsum(ex, dst, n)
    agg = _segsum(h[src] * ex[:, None], dst, n)
    return agg / (den + 1e-16)[:, None] + b


def _graphnorm(x, w, b, ms, batch, nb):
    cnt = jnp.maximum(_segsum(jnp.ones((x.shape[0],), jnp.float32), batch, nb), 1.0)
    mean = _segsum(x, batch, nb) / cnt[:, None]
    out = x - mean[batch] * ms
    var = _segsum(out * out, batch, nb) / cnt[:, None]
    return w * out / jnp.sqrt(var + 1e-5)[batch] + b


def _arma(x, Wi, Wr, b, src, dst, n, dis):
    xw = jnp.dot(x, Wi) * dis[:, None]
    prop = _segsum(xw[src], dst, n) * dis[:, None]
    return jax.nn.relu(prop + jnp.dot(x, Wr) + b)


def kernel(x, edge_index, batch, W1, as1, ad1, b1, W2, as2, ad2, b2, W3, as3, ad3, b3, gn1w, gn1b, gn1m, gn2w, gn2b, gn2m, gn3w, gn3b, gn3m, gn4w, gn4b, gn4m, gn5w, gn5b, gn5m, A1Wi, A1Wr, A1b, A2Wi, A2Wr, A2b, Wl, bl):
    n = x.shape[0]
    sl = jnp.arange(n, dtype=edge_index.dtype)
    src_sl = jnp.concatenate([edge_index[0], sl])
    dst_sl = jnp.concatenate([edge_index[1], sl])
    src = edge_index[0]
    dst = edge_index[1]

    h = jax.nn.elu(_gat(x, W1, as1, ad1, b1, src_sl, dst_sl, n))
    h = _graphnorm(h, gn1w, gn1b, gn1m, batch, NB)
    h = jax.nn.elu(_gat(h, W2, as2, ad2, b2, src_sl, dst_sl, n))
    h = _graphnorm(h, gn2w, gn2b, gn2m, batch, NB)
    h = jax.nn.elu(_gat(h, W3, as3, ad3, b3, src_sl, dst_sl, n))
    h = _graphnorm(h, gn3w, gn3b, gn3m, batch, NB)

    deg = _segsum(jnp.ones((src.shape[0],), jnp.float32), dst, n)
    dis = jnp.where(deg > 0, 1.0 / jnp.sqrt(jnp.maximum(deg, 1.0)), 0.0)
    h = jax.nn.elu(_arma(h, A1Wi, A1Wr, A1b, src, dst, n, dis))
    h = _graphnorm(h, gn4w, gn4b, gn4m, batch, NB)
    h = jax.nn.elu(_arma(h, A2Wi, A2Wr, A2b, src, dst, n, dis))
    h = _graphnorm(h, gn5w, gn5b, gn5m, batch, NB)

    return _pool_readout(h, batch, Wl, bl)
